# baseline (speedup 1.0000x reference)
import functools

import jax
import jax.numpy as jnp
from jax import lax
from jax.experimental import pallas as pl
from jax.experimental.pallas import tpu as pltpu
from jax.experimental.pallas import tpu_sc as plsc

NUM_CAT = 26
NUM_SCALAR = 16
VOCAB = 100000
ROWS = VOCAB + 1
EMB = 32
FEAT = NUM_CAT + NUM_SCALAR

B0 = 1024
B1 = 50
N = B0 * B1

NC = 2
NS = 16
NW = NC * NS
TW = N // NW

T = 64
R = NUM_CAT * T
GB = 128
NG = R // GB
CHUNKS = TW // T
S16 = R // 16


def _proj_body(x_ref, wt_ref, b_ref, o_ref):
    xs = x_ref[:, NUM_CAT:FEAT]
    o_ref[...] = (
        jnp.dot(xs, wt_ref[...], preferred_element_type=jnp.float32) + b_ref[...]
    )


_BT = 2048


def _proj_tc(xf, wt, b2):
    return pl.pallas_call(
        _proj_body,
        grid=(N // _BT,),
        in_specs=[
            pl.BlockSpec((_BT, FEAT), lambda i: (i, 0)),
            pl.BlockSpec((NUM_SCALAR, EMB), lambda i: (0, 0)),
            pl.BlockSpec((1, EMB), lambda i: (0, 0)),
        ],
        out_specs=pl.BlockSpec((_BT, EMB), lambda i: (i, 0)),
        out_shape=jax.ShapeDtypeStruct((N, EMB), jnp.float32),
    )(xf, wt, b2)


_mesh = plsc.VectorSubcoreMesh(core_axis_name="c", subcore_axis_name="s")
_cp = pltpu.CompilerParams(use_tc_tiling_on_sc=False, needs_layout_passes=False)


@functools.partial(
    pl.kernel,
    out_type=jax.ShapeDtypeStruct((N, EMB), jnp.float32),
    mesh=_mesh,
    compiler_params=_cp,
    scratch_types=[
        pltpu.VMEM((T * FEAT,), jnp.float32),
        pltpu.VMEM((R,), jnp.int32),
        pltpu.VMEM((R, EMB), jnp.float32),
        pltpu.VMEM((T, EMB), jnp.float32),
        pltpu.SemaphoreType.DMA,
    ],
)
def _sc_lookup(xflat_hbm, tab_hbm, proj_hbm, out_hbm, xbuf, gidx, gbuf, obuf, gsem):
    wid = lax.axis_index("s") * NC + lax.axis_index("c")
    wbase = wid * TW

    def chunk_body(c, _):
        tok0 = wbase + c * T

        pltpu.sync_copy(xflat_hbm.at[pl.ds(tok0 * FEAT, T * FEAT)], xbuf)

        def idx_body(s, _):
            r0 = s * 16
            rv = lax.iota(jnp.int32, 16) + r0
            q = rv // NUM_CAT
            cat = rv - q * NUM_CAT
            xv = plsc.load_gather(xbuf, [q * FEAT + cat])
            gi = xv.astype(jnp.int32) + cat * ROWS
            gidx[pl.ds(r0, 16)] = gi
            return 0

        lax.fori_loop(0, S16, idx_body, 0, unroll=4)

        handles = [
            pltpu.async_copy(
                tab_hbm.at[gidx.at[pl.ds(j * GB, GB)]],
                gbuf.at[pl.ds(j * GB, GB)],
                gsem,
            )
            for j in range(NG)
        ]

        pltpu.sync_copy(proj_hbm.at[pl.ds(tok0, T)], obuf)

        for h in handles:
            h.wait()

        def acc_body(t, _):
            r0 = t * NUM_CAT
            v0 = obuf[t, pl.ds(0, 16)]
            v1 = obuf[t, pl.ds(16, 16)]
            for i in range(NUM_CAT):
                v0 = v0 + gbuf[r0 + i, pl.ds(0, 16)]
                v1 = v1 + gbuf[r0 + i, pl.ds(16, 16)]
            obuf[t, pl.ds(0, 16)] = v0
            obuf[t, pl.ds(16, 16)] = v1
            return 0

        lax.fori_loop(0, T, acc_body, 0)

        pltpu.sync_copy(obuf, out_hbm.at[pl.ds(tok0, T)])
        return 0

    lax.fori_loop(0, CHUNKS, chunk_body, 0)


def kernel(x, tables, W, b):
    xf = x.reshape(N, FEAT)
    proj = _proj_tc(xf, W.T, b.reshape(1, EMB))
    tabf = tables.reshape(NUM_CAT * ROWS, EMB)
    out = _sc_lookup(xf.reshape(N * FEAT), tabf, proj)
    return out.reshape(B0, B1, EMB)


# speedup vs baseline: 5.5731x; 5.5731x over previous
import functools

import jax
import jax.numpy as jnp
from jax import lax
from jax.experimental import pallas as pl
from jax.experimental.pallas import tpu as pltpu
from jax.experimental.pallas import tpu_sc as plsc

NUM_CAT = 26
NUM_SCALAR = 16
VOCAB = 100000
ROWS = VOCAB + 1
EMB = 32
FEAT = NUM_CAT + NUM_SCALAR

B0 = 1024
B1 = 50
N = B0 * B1

NC = 2
NS = 16
NW = NC * NS
TW = N // NW

VB = 2048
VSTEPS = 50
BR = VB // 4
PT = VSTEPS * BR
NP = NUM_CAT * PT

PV = 4 * PT

T = 64
R = NUM_CAT * T
GB = 128
NG = R // GB
CHUNKS = TW // T
S16 = R // 16


def _pack_body(t_ref, o_ref):
    xin = t_ref[0]
    o_ref[...] = xin.reshape(EMB, BR, 4).transpose(1, 2, 0).reshape(BR, 128)


def _pack_tc(tt):
    return pl.pallas_call(
        _pack_body,
        grid=(NUM_CAT, VSTEPS),
        in_specs=[pl.BlockSpec((1, EMB, VB), lambda i, j: (i, 0, j))],
        out_specs=pl.BlockSpec((BR, 128), lambda i, j: (i * VSTEPS + j, 0)),
        out_shape=jax.ShapeDtypeStruct((NP, 128), jnp.float32),
    )(tt)


def _proj_body(x_ref, wt_ref, b_ref, o_ref):
    xs = x_ref[:, NUM_CAT:FEAT]
    o_ref[...] = (
        jnp.dot(xs, wt_ref[...], preferred_element_type=jnp.float32) + b_ref[...]
    )


_BT = 2048


def _proj_tc(xf, wt, b2):
    return pl.pallas_call(
        _proj_body,
        grid=(N // _BT,),
        in_specs=[
            pl.BlockSpec((_BT, FEAT), lambda i: (i, 0)),
            pl.BlockSpec((NUM_SCALAR, EMB), lambda i: (0, 0)),
            pl.BlockSpec((1, EMB), lambda i: (0, 0)),
        ],
        out_specs=pl.BlockSpec((_BT, EMB), lambda i: (i, 0)),
        out_shape=jax.ShapeDtypeStruct((N, EMB), jnp.float32),
    )(xf, wt, b2)


_mesh = plsc.VectorSubcoreMesh(core_axis_name="c", subcore_axis_name="s")
_cp = pltpu.CompilerParams(use_tc_tiling_on_sc=False, needs_layout_passes=False)


@functools.partial(
    pl.kernel,
    out_type=jax.ShapeDtypeStruct((N, EMB), jnp.float32),
    mesh=_mesh,
    compiler_params=_cp,
    scratch_types=[
        pltpu.VMEM((T * FEAT,), jnp.float32),
        pltpu.VMEM((R,), jnp.int32),
        pltpu.VMEM((R, EMB), jnp.float32),
        pltpu.VMEM((T, EMB), jnp.float32),
        pltpu.SemaphoreType.DMA,
    ],
)
def _sc_lookup(xflat_hbm, tab_hbm, proj_hbm, out_hbm, xbuf, gidx, gbuf, obuf, gsem):
    wid = lax.axis_index("s") * NC + lax.axis_index("c")
    wbase = wid * TW

    def chunk_body(c, _):
        tok0 = wbase + c * T

        pltpu.sync_copy(xflat_hbm.at[pl.ds(tok0 * FEAT, T * FEAT)], xbuf)

        def idx_body(s, _):
            r0 = s * 16
            rv = lax.iota(jnp.int32, 16) + r0
            q = rv // NUM_CAT
            cat = rv - q * NUM_CAT
            xv = plsc.load_gather(xbuf, [q * FEAT + cat])
            gidx[pl.ds(r0, 16)] = xv.astype(jnp.int32) + cat * PV
            return 0

        lax.fori_loop(0, S16, idx_body, 0, unroll=4)

        handles = [
            pltpu.async_copy(
                tab_hbm.at[gidx.at[pl.ds(j * GB, GB)]],
                gbuf.at[pl.ds(j * GB, GB)],
                gsem,
            )
            for j in range(NG)
        ]

        pltpu.sync_copy(proj_hbm.at[pl.ds(tok0, T)], obuf)

        for h in handles:
            h.wait()

        def acc_body(t, _):
            r0 = t * NUM_CAT
            v0 = obuf[t, pl.ds(0, 16)]
            v1 = obuf[t, pl.ds(16, 16)]
            for i in range(NUM_CAT):
                v0 = v0 + gbuf[r0 + i, pl.ds(0, 16)]
                v1 = v1 + gbuf[r0 + i, pl.ds(16, 16)]
            obuf[t, pl.ds(0, 16)] = v0
            obuf[t, pl.ds(16, 16)] = v1
            return 0

        lax.fori_loop(0, T, acc_body, 0)

        pltpu.sync_copy(obuf, out_hbm.at[pl.ds(tok0, T)])
        return 0

    lax.fori_loop(0, CHUNKS, chunk_body, 0)


def kernel(x, tables, W, b):
    tt = jnp.transpose(tables, (0, 2, 1))
    ttp = jnp.pad(tt, ((0, 0), (0, 0), (0, PV - ROWS)))
    packed = (
        ttp.reshape(NUM_CAT, EMB, PT, 4)
        .transpose(0, 2, 3, 1)
        .reshape(NP, 128)
    )
    tabf = packed.reshape(NP * 4, EMB)
    xf = x.reshape(N, FEAT)
    proj = _proj_tc(xf, W.T, b.reshape(1, EMB))
    out = _sc_lookup(xf.reshape(N * FEAT), tabf, proj)
    return out.reshape(B0, B1, EMB)
